# baseline (speedup 1.0000x reference)
import jax
import jax.numpy as jnp
from jax.experimental import pallas as pl

B, M, NEIGH, DIM = 10000, 1, 32, 128
TB = 512


def _agg_kernel(sv_ref, nv_ref, rel_ref, ue_ref, wt_ref, bias_ref, out_ref):
    rel = rel_ref[...]
    ue = ue_ref[...]
    scores = jnp.sum(rel * ue, axis=-1)
    m = jnp.max(scores, axis=-1, keepdims=True)
    e = jnp.exp(scores - m)
    w = e / jnp.sum(e, axis=-1, keepdims=True)
    nv = nv_ref[...]
    agg = jnp.sum(w[:, :, None] * nv, axis=1)
    x = sv_ref[...] + agg
    y = jnp.dot(x, wt_ref[...], preferred_element_type=jnp.float32)
    out_ref[...] = jnp.maximum(y + bias_ref[...], 0.0)


@jax.jit
def kernel(self_vectors, neighbor_vectors, neighbor_relations, user_embeddings, W, b):
    nb = self_vectors.shape[0]
    sv = self_vectors.reshape(nb, DIM)
    nv = neighbor_vectors.reshape(nb, NEIGH, DIM)
    rel = neighbor_relations.reshape(nb, NEIGH, DIM)
    ue = user_embeddings.reshape(nb, NEIGH, DIM)
    wt = W.T
    bias = b.reshape(1, DIM)

    grid = (pl.cdiv(nb, TB),)
    out = pl.pallas_call(
        _agg_kernel,
        grid=grid,
        in_specs=[
            pl.BlockSpec((TB, DIM), lambda i: (i, 0)),
            pl.BlockSpec((TB, NEIGH, DIM), lambda i: (i, 0, 0)),
            pl.BlockSpec((TB, NEIGH, DIM), lambda i: (i, 0, 0)),
            pl.BlockSpec((TB, NEIGH, DIM), lambda i: (i, 0, 0)),
            pl.BlockSpec((DIM, DIM), lambda i: (0, 0)),
            pl.BlockSpec((1, DIM), lambda i: (0, 0)),
        ],
        out_specs=pl.BlockSpec((TB, DIM), lambda i: (i, 0)),
        out_shape=jax.ShapeDtypeStruct((nb, DIM), jnp.float32),
    )(sv, nv, rel, ue, wt, bias)
    return out.reshape(nb, M, DIM)
